# baseline (speedup 1.0000x reference)
import jax
import jax.numpy as jnp
from jax import lax
from jax.experimental import pallas as pl
from jax.experimental.pallas import tpu as pltpu
from jax.experimental.pallas import tpu_sc as plsc

NC = 2
NS = 16
L = 16
NW = NC * NS
B = 16384
D = 64
BPW = B // NW
CHUNK = 128
NCH = BPW // CHUNK
VPR = D // L


def _transe_body(ent, rel, heads, rels, tails, out,
                 hidx, ridx, tidx, hbuf, rbuf, tbuf, sem):
    wid = lax.axis_index("s") * NC + lax.axis_index("c")
    pltpu.sync_copy(heads.at[wid], hidx)
    pltpu.sync_copy(rels.at[wid], ridx)
    pltpu.sync_copy(tails.at[wid], tidx)
    copies = []
    for j in range(NCH):
        dst = pl.ds(j * CHUNK, CHUNK)
        copies.append(pltpu.async_copy(ent.at[hidx.at[j]], hbuf.at[dst], sem))
        copies.append(pltpu.async_copy(rel.at[ridx.at[j]], rbuf.at[dst], sem))
        copies.append(pltpu.async_copy(ent.at[tidx.at[j]], tbuf.at[dst], sem))
    for c in copies:
        c.wait()

    @pl.loop(0, BPW)
    def _row(i):
        for k in range(VPR):
            sl = pl.ds(k * L, L)
            hbuf[i, sl] = hbuf[i, sl] + rbuf[i, sl] - tbuf[i, sl]

    pltpu.sync_copy(hbuf, out.at[pl.ds(wid * BPW, BPW)])


def kernel(entity_emb, relation_emb, heads, relations, tails):
    heads3 = heads.astype(jnp.int32).reshape(NW, NCH, CHUNK)
    rels3 = relations.astype(jnp.int32).reshape(NW, NCH, CHUNK)
    tails3 = tails.astype(jnp.int32).reshape(NW, NCH, CHUNK)
    mesh = plsc.VectorSubcoreMesh(core_axis_name="c", subcore_axis_name="s",
                                  num_cores=NC, num_subcores=NS)
    run = pl.kernel(
        _transe_body,
        out_type=jax.ShapeDtypeStruct((B, D), jnp.float32),
        mesh=mesh,
        scratch_types=[
            pltpu.VMEM((NCH, CHUNK), jnp.int32),
            pltpu.VMEM((NCH, CHUNK), jnp.int32),
            pltpu.VMEM((NCH, CHUNK), jnp.int32),
            pltpu.VMEM((BPW, D), jnp.float32),
            pltpu.VMEM((BPW, D), jnp.float32),
            pltpu.VMEM((BPW, D), jnp.float32),
            pltpu.SemaphoreType.DMA,
        ],
        compiler_params=pltpu.CompilerParams(use_tc_tiling_on_sc=False),
    )
    return run(entity_emb, relation_emb, heads3, rels3, tails3)


# speedup vs baseline: 2.4712x; 2.4712x over previous
import jax
import jax.numpy as jnp
from jax import lax
from jax.experimental import pallas as pl
from jax.experimental.pallas import tpu as pltpu
from jax.experimental.pallas import tpu_sc as plsc

NC = 2
NS = 16
L = 16
NW = NC * NS
B = 16384
D = 64
E = 1000000
R = 1000
BPW = B // NW
G = 32
GT = G // 8
NGR = BPW // G
VPR = D // L


def _transe_body(ent, rel, heads, rels, tails, out,
                 th, tr, tt, hb, rb, tb, ob, gs0, gs1, os0, os1):
    gsems = (gs0, gs1)
    osems = (os0, os1)
    wid = lax.axis_index("s") * NC + lax.axis_index("c")
    base = wid * BPW
    pltpu.sync_copy(heads.at[pl.ds(base, BPW)], th)
    pltpu.sync_copy(rels.at[pl.ds(base, BPW)], tr)
    pltpu.sync_copy(tails.at[pl.ds(base, BPW)], tt)

    def fire(g, b):
        for jj in range(G // L):
            sl = pl.ds(g * G + jj * L, L)
            vh = th[sl]
            vr = tr[sl]
            vt = tt[sl]
            for j in range(L):
                row = jj * L + j
                dst = (row // 8, row % 8)
                for vec, tab, buf in ((vh, ent, hb), (vr, rel, rb), (vt, ent, tb)):
                    s = vec[j]
                    ti = lax.shift_right_logical(s, 3)
                    ri = lax.bitwise_and(s, 7)
                    pltpu.async_copy(tab.at[ti, ri], buf.at[b, dst[0], dst[1]],
                                     gsems[b])

    def drain_gathers(b):
        for buf in (hb, rb, tb):
            pltpu.make_async_copy(ent.at[pl.ds(0, GT)], buf.at[b], gsems[b]).wait()

    def drain_out(b):
        pltpu.make_async_copy(ent.at[pl.ds(0, GT)], ob.at[b], osems[b]).wait()

    def compute(g, b):
        @pl.loop(0, G)
        def _row(r):
            t0 = r // 8
            r0 = r % 8
            for k in range(VPR):
                sl = pl.ds(k * L, L)
                ob[b, t0, r0, sl] = (hb[b, t0, r0, sl] + rb[b, t0, r0, sl]
                                     - tb[b, t0, r0, sl])

    fire(0, 0)
    fire(1, 1)

    @pl.loop(0, NGR, step=2)
    def _group(g0):
        for b in range(2):
            g = g0 + b
            drain_gathers(b)

            @pl.when(g >= 2)
            def _():
                drain_out(b)

            compute(g, b)
            pltpu.async_copy(ob.at[b], out.at[pl.ds(base // 8 + g * GT, GT)],
                             osems[b])

            @pl.when(g + 2 < NGR)
            def _():
                fire(g + 2, b)

    drain_out(0)
    drain_out(1)


def kernel(entity_emb, relation_emb, heads, relations, tails):
    ent3 = entity_emb.reshape(E // 8, 8, D)
    rel3 = relation_emb.reshape(R // 8, 8, D)
    mesh = plsc.VectorSubcoreMesh(core_axis_name="c", subcore_axis_name="s",
                                  num_cores=NC, num_subcores=NS)
    idx = pltpu.VMEM((BPW,), jnp.int32)
    buf = pltpu.VMEM((2, GT, 8, D), jnp.float32)
    run = pl.kernel(
        _transe_body,
        out_type=jax.ShapeDtypeStruct((B // 8, 8, D), jnp.float32),
        mesh=mesh,
        scratch_types=[idx, idx, idx, buf, buf, buf, buf,
                       pltpu.SemaphoreType.DMA, pltpu.SemaphoreType.DMA,
                       pltpu.SemaphoreType.DMA, pltpu.SemaphoreType.DMA],
        compiler_params=pltpu.CompilerParams(use_tc_tiling_on_sc=True,
                                             needs_layout_passes=False),
    )
    out3 = run(ent3, rel3, heads.astype(jnp.int32), relations.astype(jnp.int32),
               tails.astype(jnp.int32))
    return out3.reshape(B, D)
